# Initial kernel scaffold; baseline (speedup 1.0000x reference)
import jax
import jax.numpy as jnp
from jax.experimental import pallas as pl


def kernel(entities_with_pad, bumps_with_pad, rel_bases, rel_deltas, rel_multiples, sample):
    raise NotImplementedError("write your pallas kernel here")


# speedup vs baseline: 1.0617x; 1.0617x over previous
import functools

import jax
import jax.numpy as jnp
from jax import lax
from jax.experimental import pallas as pl
from jax.experimental.pallas import tpu as pltpu
from jax.experimental.pallas import tpu_sc as plsc

B = 4096
D = 128
NC, NS, L = 2, 16, 16
NW = NC * NS
BW = B // NW
CH = 64
NCHUNK = BW // CH
NG = CH // L


def _body(ent, bmp, relb, reld, s0, s1, sr, out,
          idx0, idx1, idxr, e0, e1, b0, b1, rb, rd, outv, sem):
    wid = lax.axis_index("s") * NC + lax.axis_index("c")
    base = wid * BW
    pltpu.sync_copy(s0.at[pl.ds(base, BW)], idx0)
    pltpu.sync_copy(s1.at[pl.ds(base, BW)], idx1)
    pltpu.sync_copy(sr.at[pl.ds(base, BW)], idxr)
    for chunk in range(NCHUNK):
        co = chunk * CH
        cps = [
            pltpu.async_copy(ent.at[idx0.at[pl.ds(co, CH)]], e0, sem),
            pltpu.async_copy(ent.at[idx1.at[pl.ds(co, CH)]], e1, sem),
            pltpu.async_copy(bmp.at[idx0.at[pl.ds(co, CH)]], b0, sem),
            pltpu.async_copy(bmp.at[idx1.at[pl.ds(co, CH)]], b1, sem),
            pltpu.async_copy(relb.at[idxr.at[pl.ds(co, CH)]], rb, sem),
            pltpu.async_copy(reld.at[idxr.at[pl.ds(co, CH)]], rd, sem),
        ]
        for cp in cps:
            cp.wait()
        for g in range(NG):
            rows = lax.iota(jnp.int32, L) + jnp.int32(g * L)

            def dim_body(dd, acc, rows=rows):
                col = jnp.full((L,), dd, jnp.int32)

                def one(a, eref, bref, acc):
                    ar = jnp.full((L,), a, jnp.int32)
                    e = plsc.load_gather(eref, [rows, col])
                    bb = plsc.load_gather(bref, [rows, col])
                    bas = plsc.load_gather(rb, [rows, ar, col])
                    dlt = plsc.load_gather(rd, [rows, ar, col])
                    pts = e + bb
                    w = jnp.abs(dlt)
                    wp = w + 1.0
                    q = 1.0 / wp
                    dist = jnp.abs(pts - bas)
                    inner = dist * q
                    outer = wp * dist - (0.5 * w) * (wp - q)
                    return acc + jnp.maximum(inner, outer)

                acc = one(0, e0, b1, acc)
                acc = one(1, e1, b0, acc)
                return acc

            acc = lax.fori_loop(0, D, dim_body, jnp.zeros((L,), jnp.float32))
            outv[pl.ds(co + g * L, L)] = acc
    pltpu.sync_copy(outv, out.at[pl.ds(base, BW)])


@functools.partial(jax.jit)
def _run(ent, bmp, relb, reld, s0, s1, sr):
    mesh = plsc.VectorSubcoreMesh(core_axis_name="c", subcore_axis_name="s")
    k = pl.kernel(
        _body,
        mesh=mesh,
        compiler_params=pltpu.CompilerParams(needs_layout_passes=False),
        out_type=jax.ShapeDtypeStruct((B,), jnp.float32),
        scratch_types=[
            pltpu.VMEM((BW,), jnp.int32),
            pltpu.VMEM((BW,), jnp.int32),
            pltpu.VMEM((BW,), jnp.int32),
            pltpu.VMEM((CH, D), jnp.float32),
            pltpu.VMEM((CH, D), jnp.float32),
            pltpu.VMEM((CH, D), jnp.float32),
            pltpu.VMEM((CH, D), jnp.float32),
            pltpu.VMEM((CH, 2, D), jnp.float32),
            pltpu.VMEM((CH, 2, D), jnp.float32),
            pltpu.VMEM((BW,), jnp.float32),
            pltpu.SemaphoreType.DMA,
        ],
    )
    return k(ent, bmp, relb, reld, s0, s1, sr)


def kernel(entities_with_pad, bumps_with_pad, rel_bases, rel_deltas,
           rel_multiples, sample):
    del rel_multiples
    s0 = sample[:, 0].astype(jnp.int32)
    s1 = sample[:, 1].astype(jnp.int32)
    sr = sample[:, 2].astype(jnp.int32)
    return _run(entities_with_pad, bumps_with_pad, rel_bases, rel_deltas,
                s0, s1, sr)


# speedup vs baseline: 3.0239x; 2.8481x over previous
import functools

import jax
import jax.numpy as jnp
from jax import lax
from jax.experimental import pallas as pl
from jax.experimental.pallas import tpu as pltpu
from jax.experimental.pallas import tpu_sc as plsc

B = 4096
D = 128
NV = D // 16
NC, NS, L = 2, 16, 16
NW = NC * NS
BW = B // NW
CH = 32
NCHUNK = BW // CH
NG = CH // L


def _body(ent, bmp, relb, reld, s0, s1, sr, out,
          idx0, idx1, idxr,
          e0a, e1a, b0a, b1a, rba, rda,
          e0b, e1b, b0b, b1b, rbb, rdb,
          accT, outv, sema, semb):
    wid = lax.axis_index("s") * NC + lax.axis_index("c")
    base = wid * BW
    pltpu.sync_copy(s0.at[pl.ds(base, BW)], idx0)
    pltpu.sync_copy(s1.at[pl.ds(base, BW)], idx1)
    pltpu.sync_copy(sr.at[pl.ds(base, BW)], idxr)

    bufs = [(e0a, e1a, b0a, b1a, rba, rda, sema),
            (e0b, e1b, b0b, b1b, rbb, rdb, semb)]

    def issue(c, s):
        co = c * CH
        e0, e1, b0, b1, rb, rd, sem = bufs[s]
        return [
            pltpu.async_copy(ent.at[idx0.at[pl.ds(co, CH)]], e0, sem),
            pltpu.async_copy(ent.at[idx1.at[pl.ds(co, CH)]], e1, sem),
            pltpu.async_copy(bmp.at[idx0.at[pl.ds(co, CH)]], b0, sem),
            pltpu.async_copy(bmp.at[idx1.at[pl.ds(co, CH)]], b1, sem),
            pltpu.async_copy(relb.at[idxr.at[pl.ds(co, CH)]], rb, sem),
            pltpu.async_copy(reld.at[idxr.at[pl.ds(co, CH)]], rd, sem),
        ]

    lanes = lax.iota(jnp.int32, L)
    cps = issue(0, 0)
    for c in range(NCHUNK):
        s = c % 2
        e0, e1, b0, b1, rb, rd, _ = bufs[s]
        for cp in cps:
            cp.wait()
        if c + 1 < NCHUNK:
            cps = issue(c + 1, 1 - s)

        def sample_body(i, carry, e0=e0, e1=e1, b0=b0, b1=b1, rb=rb, rd=rd):
            acc = jnp.zeros((L,), jnp.float32)
            for a in range(2):
                eref, bref = (e0, b1) if a == 0 else (e1, b0)
                for v in range(NV):
                    sl = pl.ds(v * L, L)
                    e = eref[i, sl]
                    bb = bref[i, sl]
                    bas = rb[i, a, sl]
                    dlt = rd[i, a, sl]
                    pts = e + bb
                    w = jnp.abs(dlt)
                    wp = w + 1.0
                    q = 1.0 / wp
                    dist = jnp.abs(pts - bas)
                    inner = dist * q
                    outer = wp * dist - (0.5 * w) * (wp - q)
                    acc = acc + jnp.maximum(inner, outer)
            col = jnp.full((L,), i, jnp.int32)
            plsc.store_scatter(accT, [lanes, col], acc)
            return carry

        lax.fori_loop(0, CH, sample_body, jnp.int32(0))

        for g in range(NG):
            acc16 = accT[0, pl.ds(g * L, L)]
            for j in range(1, L):
                acc16 = acc16 + accT[j, pl.ds(g * L, L)]
            outv[pl.ds(c * CH + g * L, L)] = acc16
    pltpu.sync_copy(outv, out.at[pl.ds(base, BW)])


@functools.partial(jax.jit)
def _run(ent, bmp, relb, reld, s0, s1, sr):
    mesh = plsc.VectorSubcoreMesh(core_axis_name="c", subcore_axis_name="s")
    ebuf = pltpu.VMEM((CH, D), jnp.float32)
    rbuf = pltpu.VMEM((CH, 2, D), jnp.float32)
    k = pl.kernel(
        _body,
        mesh=mesh,
        compiler_params=pltpu.CompilerParams(needs_layout_passes=False),
        out_type=jax.ShapeDtypeStruct((B,), jnp.float32),
        scratch_types=[
            pltpu.VMEM((BW,), jnp.int32),
            pltpu.VMEM((BW,), jnp.int32),
            pltpu.VMEM((BW,), jnp.int32),
            ebuf, ebuf, ebuf, ebuf, rbuf, rbuf,
            ebuf, ebuf, ebuf, ebuf, rbuf, rbuf,
            pltpu.VMEM((L, CH), jnp.float32),
            pltpu.VMEM((BW,), jnp.float32),
            pltpu.SemaphoreType.DMA,
            pltpu.SemaphoreType.DMA,
        ],
    )
    return k(ent, bmp, relb, reld, s0, s1, sr)


def kernel(entities_with_pad, bumps_with_pad, rel_bases, rel_deltas,
           rel_multiples, sample):
    del rel_multiples
    s0 = sample[:, 0].astype(jnp.int32)
    s1 = sample[:, 1].astype(jnp.int32)
    sr = sample[:, 2].astype(jnp.int32)
    return _run(entities_with_pad, bumps_with_pad, rel_bases, rel_deltas,
                s0, s1, sr)


# speedup vs baseline: 3.1259x; 1.0337x over previous
import functools

import jax
import jax.numpy as jnp
from jax import lax
from jax.experimental import pallas as pl
from jax.experimental.pallas import tpu as pltpu
from jax.experimental.pallas import tpu_sc as plsc

B = 4096
D = 128
NV = D // 16
NC, NS, L = 2, 16, 16
NW = NC * NS
BW = B // NW
CH = 32
NCHUNK = BW // CH
NG = CH // L
CT = CH + 1


def _body(ent, bmp, relb, reld, smp, out,
          smpv, idx01, idxr,
          eba, rba, rda,
          ebb, rbb, rdb,
          accT, outv, sema, semb):
    wid = lax.axis_index("s") * NC + lax.axis_index("c")
    base = wid * BW
    pltpu.sync_copy(smp.at[pl.ds(base, BW)], smpv)

    lanes = lax.iota(jnp.int32, L)
    for g in range(BW // L):
        rows16 = lanes + jnp.int32(g * L)
        c, h = g // NG, g % NG
        i0 = plsc.load_gather(smpv, [rows16, jnp.full((L,), 0, jnp.int32)])
        i1 = plsc.load_gather(smpv, [rows16, jnp.full((L,), 1, jnp.int32)])
        ir = plsc.load_gather(smpv, [rows16, jnp.full((L,), 2, jnp.int32)])
        idx01[pl.ds(c * 2 * CH + h * L, L)] = i0
        idx01[pl.ds(c * 2 * CH + CH + h * L, L)] = i1
        idxr[pl.ds(g * L, L)] = ir

    bufs = [(eba, rba, rda, sema), (ebb, rbb, rdb, semb)]

    def issue(c, s):
        eb, rb, rd, sem = bufs[s]
        return [
            pltpu.async_copy(ent.at[idx01.at[pl.ds(c * 2 * CH, 2 * CH)]],
                             eb.at[0], sem),
            pltpu.async_copy(bmp.at[idx01.at[pl.ds(c * 2 * CH, 2 * CH)]],
                             eb.at[1], sem),
            pltpu.async_copy(relb.at[idxr.at[pl.ds(c * CH, CH)]], rb, sem),
            pltpu.async_copy(reld.at[idxr.at[pl.ds(c * CH, CH)]], rd, sem),
        ]

    cps = issue(0, 0)
    for c in range(NCHUNK):
        s = c % 2
        eb, rb, rd, _ = bufs[s]
        for cp in cps:
            cp.wait()
        if c + 1 < NCHUNK:
            cps = issue(c + 1, 1 - s)

        def sample_body(i, carry, eb=eb, rb=rb, rd=rd):
            acc = jnp.zeros((L,), jnp.float32)
            for a in range(2):
                for v in range(NV):
                    sl = pl.ds(v * L, L)
                    e = eb[0, i + a * CH, sl]
                    bb = eb[1, i + (1 - a) * CH, sl]
                    bas = rb[i, a, sl]
                    dlt = rd[i, a, sl]
                    pts = e + bb
                    w = jnp.abs(dlt)
                    wp = w + 1.0
                    q = 1.0 / wp
                    dist = jnp.abs(pts - bas)
                    inner = dist * q
                    outer = wp * dist - (0.5 * w) * (wp - q)
                    acc = acc + jnp.maximum(inner, outer)
            col = jnp.full((L,), i, jnp.int32)
            plsc.store_scatter(accT, [lanes, col], acc)
            return carry

        lax.fori_loop(0, CH, sample_body, jnp.int32(0))

        for g in range(NG):
            acc16 = accT[0, pl.ds(g * L, L)]
            for j in range(1, L):
                acc16 = acc16 + accT[j, pl.ds(g * L, L)]
            outv[pl.ds(c * CH + g * L, L)] = acc16
    pltpu.sync_copy(outv, out.at[pl.ds(base, BW)])


@functools.partial(jax.jit)
def _run(ent, bmp, relb, reld, smp):
    mesh = plsc.VectorSubcoreMesh(core_axis_name="c", subcore_axis_name="s")
    ebuf = pltpu.VMEM((2, 2 * CH, D), jnp.float32)
    rbuf = pltpu.VMEM((CH, 2, D), jnp.float32)
    k = pl.kernel(
        _body,
        mesh=mesh,
        compiler_params=pltpu.CompilerParams(needs_layout_passes=False),
        out_type=jax.ShapeDtypeStruct((B,), jnp.float32),
        scratch_types=[
            pltpu.VMEM((BW, 3), jnp.int32),
            pltpu.VMEM((2 * BW,), jnp.int32),
            pltpu.VMEM((BW,), jnp.int32),
            ebuf, rbuf, rbuf,
            ebuf, rbuf, rbuf,
            pltpu.VMEM((L, CT), jnp.float32),
            pltpu.VMEM((BW,), jnp.float32),
            pltpu.SemaphoreType.DMA,
            pltpu.SemaphoreType.DMA,
        ],
    )
    return k(ent, bmp, relb, reld, smp)


def kernel(entities_with_pad, bumps_with_pad, rel_bases, rel_deltas,
           rel_multiples, sample):
    del rel_multiples
    return _run(entities_with_pad, bumps_with_pad, rel_bases, rel_deltas,
                sample.astype(jnp.int32))
